# baseline (speedup 1.0000x reference)
import jax
import jax.numpy as jnp
from jax.experimental import pallas as pl

H = 4
DH = 32
HID = 128
NA = 25000
NB = 25000

BLK = 1000
PW = 256


def _leaky(x, slope):
    return jnp.where(x > 0, x, slope * x)


def _proj_body(x_ref, w_ref, o_ref):
    o_ref[...] = jnp.dot(x_ref[...], w_ref[...],
                         preferred_element_type=jnp.float32,
                         precision=jax.lax.Precision.HIGHEST)


def _project(x, wbig):
    n = x.shape[0]
    return pl.pallas_call(
        _proj_body,
        grid=(n // BLK,),
        in_specs=[pl.BlockSpec((BLK, HID), lambda i: (i, 0)),
                  pl.BlockSpec((HID, PW), lambda i: (0, 0))],
        out_specs=pl.BlockSpec((BLK, PW), lambda i: (i, 0)),
        out_shape=jax.ShapeDtypeStruct((n, PW), jnp.float32),
    )(x, wbig)


def _finalize_body(num_ref, den_ref, bias_ref, o_ref, ps_ref, pq_ref):
    den = den_ref[...]
    o = jnp.where(den > 0, num_ref[...] / den, 0.0) + bias_ref[...]
    o_ref[...] = o
    ps_ref[...] = jnp.sum(o, axis=0, keepdims=True)[None]
    pq_ref[...] = jnp.sum(o * o, axis=0, keepdims=True)[None]


def _finalize(num, den_rep, bias_row):
    n = num.shape[0]
    g = n // BLK
    return pl.pallas_call(
        _finalize_body,
        grid=(g,),
        in_specs=[pl.BlockSpec((BLK, HID), lambda i: (i, 0)),
                  pl.BlockSpec((BLK, HID), lambda i: (i, 0)),
                  pl.BlockSpec((1, HID), lambda i: (0, 0))],
        out_specs=(pl.BlockSpec((BLK, HID), lambda i: (i, 0)),
                   pl.BlockSpec((1, 1, HID), lambda i: (i, 0, 0)),
                   pl.BlockSpec((1, 1, HID), lambda i: (i, 0, 0))),
        out_shape=(jax.ShapeDtypeStruct((n, HID), jnp.float32),
                   jax.ShapeDtypeStruct((g, 1, HID), jnp.float32),
                   jax.ShapeDtypeStruct((g, 1, HID), jnp.float32)),
    )(num, den_rep, bias_row)


def _affine_tanh_body(x_ref, s_ref, t_ref, o_ref):
    o_ref[...] = jnp.tanh(x_ref[...] * s_ref[...] + t_ref[...])


def _affine_leaky_body(x_ref, s_ref, t_ref, o_ref):
    y = x_ref[...] * s_ref[...] + t_ref[...]
    o_ref[...] = jnp.where(y > 0, y, 0.01 * y)


def _affine_act(x, scale_row, shift_row, use_tanh):
    n = x.shape[0]
    body = _affine_tanh_body if use_tanh else _affine_leaky_body
    return pl.pallas_call(
        body,
        grid=(n // BLK,),
        in_specs=[pl.BlockSpec((BLK, HID), lambda i: (i, 0)),
                  pl.BlockSpec((1, HID), lambda i: (0, 0)),
                  pl.BlockSpec((1, HID), lambda i: (0, 0))],
        out_specs=pl.BlockSpec((BLK, HID), lambda i: (i, 0)),
        out_shape=jax.ShapeDtypeStruct((n, HID), jnp.float32),
    )(x, scale_row, shift_row)


def _attn_mat(a):
    return (jnp.eye(H, dtype=a.dtype)[:, None, :] * a[:, :, None]).reshape(HID, H)


def _edge_aggregate(fs, el, er, src, dst, n_dst):
    e = _leaky(el[src] + er[dst], 0.2)
    w = jnp.exp(e)
    w_rep = jnp.repeat(w, DH, axis=1)
    msg = jnp.concatenate([w_rep * fs[src], w], axis=1)
    seg = jax.ops.segment_sum(msg, dst, num_segments=n_dst)
    num = seg[:, :HID]
    den_rep = jnp.repeat(seg[:, HID:], DH, axis=1)
    return num, den_rep


def kernel(x_a, x_b, ab_src, ab_dst, ba_src, ba_dst, W, attn_l, attn_r,
           bias, gamma_a, beta_a, gamma_b, beta_b):
    wbig_a, wbig_b = [], []
    for i in range(4):
        wal_ab = W[i, 0] @ _attn_mat(attn_l[i, 0])
        war_ab = W[i, 0] @ _attn_mat(attn_r[i, 0])
        wal_ba = W[i, 1] @ _attn_mat(attn_l[i, 1])
        war_ba = W[i, 1] @ _attn_mat(attn_r[i, 1])
        pad = jnp.zeros((HID, PW - HID - 2 * H), jnp.float32)
        wbig_a.append(jnp.concatenate([W[i, 0], wal_ab, war_ba, pad], axis=1))
        wbig_b.append(jnp.concatenate([W[i, 1], wal_ba, war_ab, pad], axis=1))

    h_a, h_b = x_a, x_b
    for i in range(4):
        pa = _project(h_a, wbig_a[i])
        pb = _project(h_b, wbig_b[i])
        fs_ab, el_ab, er_ba = pa[:, :HID], pa[:, HID:HID + H], pa[:, HID + H:HID + 2 * H]
        fs_ba, el_ba, er_ab = pb[:, :HID], pb[:, HID:HID + H], pb[:, HID + H:HID + 2 * H]

        num_b, den_b = _edge_aggregate(fs_ab, el_ab, er_ab, ab_src, ab_dst, NB)
        num_a, den_a = _edge_aggregate(fs_ba, el_ba, er_ba, ba_src, ba_dst, NA)

        nb_out, ps_b, pq_b = _finalize(num_b, den_b, bias[i, 0].reshape(1, HID))
        na_out, ps_a, pq_a = _finalize(num_a, den_a, bias[i, 1].reshape(1, HID))
        h_a, h_b = na_out, nb_out

        if i < 3:
            mu_a = jnp.sum(ps_a, axis=(0, 1), keepdims=False)[None] / NA
            var_a = jnp.sum(pq_a, axis=(0, 1), keepdims=False)[None] / NA - mu_a * mu_a
            mu_b = jnp.sum(ps_b, axis=(0, 1), keepdims=False)[None] / NB
            var_b = jnp.sum(pq_b, axis=(0, 1), keepdims=False)[None] / NB - mu_b * mu_b
            sc_a = gamma_a[i][None, :] / jnp.sqrt(var_a + 1e-5)
            sh_a = beta_a[i][None, :] - mu_a * sc_a
            sc_b = gamma_b[i][None, :] / jnp.sqrt(var_b + 1e-5)
            sh_b = beta_b[i][None, :] - mu_b * sc_b
            h_a = _affine_act(h_a, sc_a, sh_a, use_tanh=(i == 2))
            h_b = _affine_act(h_b, sc_b, sh_b, use_tanh=(i == 2))

    return jnp.stack([h_a, h_b])


# speedup vs baseline: 1.0685x; 1.0685x over previous
import jax
import jax.numpy as jnp
from jax.experimental import pallas as pl

H = 4
DH = 32
HID = 128
NA = 25000
NB = 25000

BLK = 1000
PW = 256


def _leaky(x, slope):
    return jnp.where(x > 0, x, slope * x)


def _proj_body(x_ref, w_ref, o_ref):
    o_ref[...] = jnp.dot(x_ref[...], w_ref[...],
                         preferred_element_type=jnp.float32,
                         precision=jax.lax.Precision.HIGHEST)


def _project(x, wbig):
    n = x.shape[0]
    return pl.pallas_call(
        _proj_body,
        grid=(n // BLK,),
        in_specs=[pl.BlockSpec((BLK, HID), lambda i: (i, 0)),
                  pl.BlockSpec((HID, PW), lambda i: (0, 0))],
        out_specs=pl.BlockSpec((BLK, PW), lambda i: (i, 0)),
        out_shape=jax.ShapeDtypeStruct((n, PW), jnp.float32),
    )(x, wbig)


def _finalize_body(num_ref, den_ref, bias_ref, o_ref, ps_ref, pq_ref):
    den = den_ref[...]
    o = jnp.where(den > 0, num_ref[...] / den, 0.0) + bias_ref[...]
    o_ref[...] = o
    ps_ref[...] = jnp.sum(o, axis=0, keepdims=True)[None]
    pq_ref[...] = jnp.sum(o * o, axis=0, keepdims=True)[None]


def _finalize(num, den_rep, bias_row):
    n = num.shape[0]
    g = n // BLK
    return pl.pallas_call(
        _finalize_body,
        grid=(g,),
        in_specs=[pl.BlockSpec((BLK, HID), lambda i: (i, 0)),
                  pl.BlockSpec((BLK, HID), lambda i: (i, 0)),
                  pl.BlockSpec((1, HID), lambda i: (0, 0))],
        out_specs=(pl.BlockSpec((BLK, HID), lambda i: (i, 0)),
                   pl.BlockSpec((1, 1, HID), lambda i: (i, 0, 0)),
                   pl.BlockSpec((1, 1, HID), lambda i: (i, 0, 0))),
        out_shape=(jax.ShapeDtypeStruct((n, HID), jnp.float32),
                   jax.ShapeDtypeStruct((g, 1, HID), jnp.float32),
                   jax.ShapeDtypeStruct((g, 1, HID), jnp.float32)),
    )(num, den_rep, bias_row)


def _affine_tanh_body(x_ref, s_ref, t_ref, o_ref):
    o_ref[...] = jnp.tanh(x_ref[...] * s_ref[...] + t_ref[...])


def _affine_leaky_body(x_ref, s_ref, t_ref, o_ref):
    y = x_ref[...] * s_ref[...] + t_ref[...]
    o_ref[...] = jnp.where(y > 0, y, 0.01 * y)


def _affine_act(x, scale_row, shift_row, use_tanh):
    n = x.shape[0]
    body = _affine_tanh_body if use_tanh else _affine_leaky_body
    return pl.pallas_call(
        body,
        grid=(n // BLK,),
        in_specs=[pl.BlockSpec((BLK, HID), lambda i: (i, 0)),
                  pl.BlockSpec((1, HID), lambda i: (0, 0)),
                  pl.BlockSpec((1, HID), lambda i: (0, 0))],
        out_specs=pl.BlockSpec((BLK, HID), lambda i: (i, 0)),
        out_shape=jax.ShapeDtypeStruct((n, HID), jnp.float32),
    )(x, scale_row, shift_row)


def _attn_mat(a):
    return (jnp.eye(H, dtype=a.dtype)[:, None, :] * a[:, :, None]).reshape(HID, H)


def _edge_aggregate(fs, el, er, src, dst, n_dst):
    e = _leaky(el[src] + er[dst], 0.2)
    w = jnp.exp(e)
    den = jax.ops.segment_sum(w, dst, num_segments=n_dst,
                              indices_are_sorted=True)
    w_rep = jnp.repeat(w, DH, axis=1)
    num = jax.ops.segment_sum(w_rep * fs[src], dst, num_segments=n_dst,
                              indices_are_sorted=True)
    den_rep = jnp.repeat(den, DH, axis=1)
    return num, den_rep


def kernel(x_a, x_b, ab_src, ab_dst, ba_src, ba_dst, W, attn_l, attn_r,
           bias, gamma_a, beta_a, gamma_b, beta_b):
    wbig_a, wbig_b = [], []
    for i in range(4):
        wal_ab = W[i, 0] @ _attn_mat(attn_l[i, 0])
        war_ab = W[i, 0] @ _attn_mat(attn_r[i, 0])
        wal_ba = W[i, 1] @ _attn_mat(attn_l[i, 1])
        war_ba = W[i, 1] @ _attn_mat(attn_r[i, 1])
        pad = jnp.zeros((HID, PW - HID - 2 * H), jnp.float32)
        wbig_a.append(jnp.concatenate([W[i, 0], wal_ab, war_ba, pad], axis=1))
        wbig_b.append(jnp.concatenate([W[i, 1], wal_ba, war_ab, pad], axis=1))

    ab_p = jnp.argsort(ab_dst)
    ab_src, ab_dst = ab_src[ab_p], ab_dst[ab_p]
    ba_p = jnp.argsort(ba_dst)
    ba_src, ba_dst = ba_src[ba_p], ba_dst[ba_p]

    h_a, h_b = x_a, x_b
    for i in range(4):
        pa = _project(h_a, wbig_a[i])
        pb = _project(h_b, wbig_b[i])
        fs_ab, el_ab, er_ba = pa[:, :HID], pa[:, HID:HID + H], pa[:, HID + H:HID + 2 * H]
        fs_ba, el_ba, er_ab = pb[:, :HID], pb[:, HID:HID + H], pb[:, HID + H:HID + 2 * H]

        num_b, den_b = _edge_aggregate(fs_ab, el_ab, er_ab, ab_src, ab_dst, NB)
        num_a, den_a = _edge_aggregate(fs_ba, el_ba, er_ba, ba_src, ba_dst, NA)

        nb_out, ps_b, pq_b = _finalize(num_b, den_b, bias[i, 0].reshape(1, HID))
        na_out, ps_a, pq_a = _finalize(num_a, den_a, bias[i, 1].reshape(1, HID))
        h_a, h_b = na_out, nb_out

        if i < 3:
            mu_a = jnp.sum(ps_a, axis=(0, 1), keepdims=False)[None] / NA
            var_a = jnp.sum(pq_a, axis=(0, 1), keepdims=False)[None] / NA - mu_a * mu_a
            mu_b = jnp.sum(ps_b, axis=(0, 1), keepdims=False)[None] / NB
            var_b = jnp.sum(pq_b, axis=(0, 1), keepdims=False)[None] / NB - mu_b * mu_b
            sc_a = gamma_a[i][None, :] / jnp.sqrt(var_a + 1e-5)
            sh_a = beta_a[i][None, :] - mu_a * sc_a
            sc_b = gamma_b[i][None, :] / jnp.sqrt(var_b + 1e-5)
            sh_b = beta_b[i][None, :] - mu_b * sc_b
            h_a = _affine_act(h_a, sc_a, sh_a, use_tanh=(i == 2))
            h_b = _affine_act(h_b, sc_b, sh_b, use_tanh=(i == 2))

    return jnp.stack([h_a, h_b])


# speedup vs baseline: 1.0759x; 1.0069x over previous
import jax
import jax.numpy as jnp
from jax.experimental import pallas as pl

H = 4
DH = 32
HID = 128
NA = 25000
NB = 25000

BLK = 1000
PW = 256


def _leaky(x, slope):
    return jnp.where(x > 0, x, slope * x)


def _proj_body(x_ref, w_ref, o_ref):
    o_ref[...] = jnp.dot(x_ref[...], w_ref[...],
                         preferred_element_type=jnp.float32,
                         precision=jax.lax.Precision.HIGHEST)


def _project(x, wbig):
    n = x.shape[0]
    return pl.pallas_call(
        _proj_body,
        grid=(n // BLK,),
        in_specs=[pl.BlockSpec((BLK, HID), lambda i: (i, 0)),
                  pl.BlockSpec((HID, PW), lambda i: (0, 0))],
        out_specs=pl.BlockSpec((BLK, PW), lambda i: (i, 0)),
        out_shape=jax.ShapeDtypeStruct((n, PW), jnp.float32),
    )(x, wbig)


def _finalize_body(num_ref, den_ref, bias_ref, o_ref, ps_ref, pq_ref):
    den = den_ref[...]
    o = jnp.where(den > 0, num_ref[...] / den, 0.0) + bias_ref[...]
    o_ref[...] = o
    ps_ref[...] = jnp.sum(o, axis=0, keepdims=True)[None]
    pq_ref[...] = jnp.sum(o * o, axis=0, keepdims=True)[None]


def _finalize(num, den_rep, bias_row):
    n = num.shape[0]
    g = n // BLK
    return pl.pallas_call(
        _finalize_body,
        grid=(g,),
        in_specs=[pl.BlockSpec((BLK, HID), lambda i: (i, 0)),
                  pl.BlockSpec((BLK, HID), lambda i: (i, 0)),
                  pl.BlockSpec((1, HID), lambda i: (0, 0))],
        out_specs=(pl.BlockSpec((BLK, HID), lambda i: (i, 0)),
                   pl.BlockSpec((1, 1, HID), lambda i: (i, 0, 0)),
                   pl.BlockSpec((1, 1, HID), lambda i: (i, 0, 0))),
        out_shape=(jax.ShapeDtypeStruct((n, HID), jnp.float32),
                   jax.ShapeDtypeStruct((g, 1, HID), jnp.float32),
                   jax.ShapeDtypeStruct((g, 1, HID), jnp.float32)),
    )(num, den_rep, bias_row)


def _affine_tanh_body(x_ref, s_ref, t_ref, o_ref):
    o_ref[...] = jnp.tanh(x_ref[...] * s_ref[...] + t_ref[...])


def _affine_leaky_body(x_ref, s_ref, t_ref, o_ref):
    y = x_ref[...] * s_ref[...] + t_ref[...]
    o_ref[...] = jnp.where(y > 0, y, 0.01 * y)


def _affine_act(x, scale_row, shift_row, use_tanh):
    n = x.shape[0]
    body = _affine_tanh_body if use_tanh else _affine_leaky_body
    return pl.pallas_call(
        body,
        grid=(n // BLK,),
        in_specs=[pl.BlockSpec((BLK, HID), lambda i: (i, 0)),
                  pl.BlockSpec((1, HID), lambda i: (0, 0)),
                  pl.BlockSpec((1, HID), lambda i: (0, 0))],
        out_specs=pl.BlockSpec((BLK, HID), lambda i: (i, 0)),
        out_shape=jax.ShapeDtypeStruct((n, HID), jnp.float32),
    )(x, scale_row, shift_row)


def _attn_mat(a):
    return (jnp.eye(H, dtype=a.dtype)[:, None, :] * a[:, :, None]).reshape(HID, H)


def _edge_aggregate(fs, el, er, src, dst, n_dst):
    e = _leaky(el[src] + er[dst], 0.2)
    w = jnp.exp(e)
    den = jax.ops.segment_sum(w, dst, num_segments=n_dst)
    w_rep = jnp.repeat(w, DH, axis=1)
    num = jax.ops.segment_sum(w_rep * fs[src], dst, num_segments=n_dst)
    den_rep = jnp.repeat(den, DH, axis=1)
    return num, den_rep


def kernel(x_a, x_b, ab_src, ab_dst, ba_src, ba_dst, W, attn_l, attn_r,
           bias, gamma_a, beta_a, gamma_b, beta_b):
    wbig_a, wbig_b = [], []
    for i in range(4):
        wal_ab = W[i, 0] @ _attn_mat(attn_l[i, 0])
        war_ab = W[i, 0] @ _attn_mat(attn_r[i, 0])
        wal_ba = W[i, 1] @ _attn_mat(attn_l[i, 1])
        war_ba = W[i, 1] @ _attn_mat(attn_r[i, 1])
        pad = jnp.zeros((HID, PW - HID - 2 * H), jnp.float32)
        wbig_a.append(jnp.concatenate([W[i, 0], wal_ab, war_ba, pad], axis=1))
        wbig_b.append(jnp.concatenate([W[i, 1], wal_ba, war_ab, pad], axis=1))

    h_a, h_b = x_a, x_b
    for i in range(4):
        pa = _project(h_a, wbig_a[i])
        pb = _project(h_b, wbig_b[i])
        fs_ab, el_ab, er_ba = pa[:, :HID], pa[:, HID:HID + H], pa[:, HID + H:HID + 2 * H]
        fs_ba, el_ba, er_ab = pb[:, :HID], pb[:, HID:HID + H], pb[:, HID + H:HID + 2 * H]

        num_b, den_b = _edge_aggregate(fs_ab, el_ab, er_ab, ab_src, ab_dst, NB)
        num_a, den_a = _edge_aggregate(fs_ba, el_ba, er_ba, ba_src, ba_dst, NA)

        nb_out, ps_b, pq_b = _finalize(num_b, den_b, bias[i, 0].reshape(1, HID))
        na_out, ps_a, pq_a = _finalize(num_a, den_a, bias[i, 1].reshape(1, HID))
        h_a, h_b = na_out, nb_out

        if i < 3:
            mu_a = jnp.sum(ps_a, axis=(0, 1), keepdims=False)[None] / NA
            var_a = jnp.sum(pq_a, axis=(0, 1), keepdims=False)[None] / NA - mu_a * mu_a
            mu_b = jnp.sum(ps_b, axis=(0, 1), keepdims=False)[None] / NB
            var_b = jnp.sum(pq_b, axis=(0, 1), keepdims=False)[None] / NB - mu_b * mu_b
            sc_a = gamma_a[i][None, :] / jnp.sqrt(var_a + 1e-5)
            sh_a = beta_a[i][None, :] - mu_a * sc_a
            sc_b = gamma_b[i][None, :] / jnp.sqrt(var_b + 1e-5)
            sh_b = beta_b[i][None, :] - mu_b * sc_b
            h_a = _affine_act(h_a, sc_a, sh_a, use_tanh=(i == 2))
            h_b = _affine_act(h_b, sc_b, sh_b, use_tanh=(i == 2))

    return jnp.stack([h_a, h_b])
